# baseline (speedup 1.0000x reference)
import jax
import jax.numpy as jnp
from jax import lax
from jax.experimental import pallas as pl
from jax.experimental.pallas import tpu as pltpu
from jax.experimental.pallas import tpu_sc as plsc

B, L, H = 4096, 200, 64
BL = B * L
NC, NS = 2, 16
NW = NC * NS
PER_W = BL // NW
CHUNK = 128
NCHUNK = PER_W // CHUNK


def _body(x_hbm, tab_hbm, pos_hbm, out_hbm, idx_v, buf, pos_rep, sem):
    wid = lax.axis_index("s") * NC + lax.axis_index("c")
    base = wid * PER_W

    pltpu.sync_copy(pos_hbm, pos_rep.at[pl.ds(0, L)])
    pltpu.sync_copy(pos_hbm, pos_rep.at[pl.ds(L, L)])

    def chunk_body(c, carry):
        gbase = base + c * CHUNK
        pltpu.sync_copy(x_hbm.at[pl.ds(gbase, CHUNK)], idx_v)
        pltpu.async_copy(tab_hbm.at[idx_v], buf, sem).wait()
        smod = lax.rem(c * CHUNK, L)

        def row_body(i, c2):
            pr = smod + i
            for j in range(H // 16):
                v = pos_rep[pr, pl.ds(16 * j, 16)]
                plsc.addupdate(buf.at[i, pl.ds(16 * j, 16)], v)
            return c2

        lax.fori_loop(0, CHUNK, row_body, 0)
        pltpu.sync_copy(buf, out_hbm.at[pl.ds(gbase, CHUNK)])
        return carry

    lax.fori_loop(0, NCHUNK, chunk_body, 0)


def kernel(x, embed_table, pos_table):
    xf = x.reshape(BL)
    mesh = plsc.VectorSubcoreMesh(core_axis_name="c", subcore_axis_name="s")
    out = pl.kernel(
        _body,
        out_type=jax.ShapeDtypeStruct((BL, H), jnp.float32),
        mesh=mesh,
        compiler_params=pltpu.CompilerParams(use_tc_tiling_on_sc=False),
        scratch_types=[
            pltpu.VMEM((CHUNK,), jnp.int32),
            pltpu.VMEM((CHUNK, H), jnp.float32),
            pltpu.VMEM((2 * L, H), jnp.float32),
            pltpu.SemaphoreType.DMA,
        ],
    )(xf, embed_table, pos_table)
    return out.reshape(B, L, H)
